# baseline (speedup 1.0000x reference)
import functools

import jax
import jax.numpy as jnp
from jax import lax
from jax.experimental import pallas as pl
from jax.experimental.pallas import tpu as pltpu
from jax.experimental.pallas import tpu_sc as plsc

N = 2048
T = 3
F = 26
H = 128
E = 16384
NM = 842
EM = 6736
NL = 122
EL = 976
NLAYERS = 4
OUT = 26

NMP = 896
NLP = 128
OFF_M = N
OFF_L = N + NMP
NP = N + NMP + NLP
EMP = 7168
ELP = 1024
EP = E + EMP + ELP

NC = 2
NS = 16
NW = NC * NS
EPW = EP // NW
KCH = EPW // 128

_f32 = jnp.float32


def _leaky(x):
    return jnp.where(x >= 0, x, 0.01 * x)


def _dot(a, b):
    return jnp.dot(a.astype(jnp.bfloat16), b.astype(jnp.bfloat16),
                   preferred_element_type=_f32)


def _padr(a, rows):
    pad = [(0, rows - a.shape[0])] + [(0, 0)] * (a.ndim - 1)
    return jnp.pad(a, pad)


def _rmlp_tc(xv, p):
    if xv.shape[1] < 8:
        pk = 128 - xv.shape[1]
        xv = jnp.pad(xv, ((0, 0), (0, pk)))
        p = dict(p,
                 l1={"W": jnp.pad(p["l1"]["W"], ((0, pk), (0, 0))),
                     "b": p["l1"]["b"]},
                 skip={"W": jnp.pad(p["skip"]["W"], ((0, pk), (0, 0))),
                       "b": p["skip"]["b"]})
    R, din = xv.shape
    h = p["l1"]["W"].shape[1]
    dout = p["l2"]["W"].shape[1]
    br = 512 if R % 512 == 0 else 128

    def body(x_ref, w1, b1, w2, b2, ws, bs, o_ref):
        xb = x_ref[...]
        hh = _leaky(_dot(xb, w1[...]) + b1[...])
        o_ref[...] = (_dot(hh, w2[...]) + b2[...]
                      + _dot(xb, ws[...]) + bs[...])

    return pl.pallas_call(
        body,
        grid=(R // br,),
        in_specs=[
            pl.BlockSpec((br, din), lambda i: (i, 0)),
            pl.BlockSpec((din, h), lambda i: (0, 0)),
            pl.BlockSpec((1, h), lambda i: (0, 0)),
            pl.BlockSpec((h, dout), lambda i: (0, 0)),
            pl.BlockSpec((1, dout), lambda i: (0, 0)),
            pl.BlockSpec((din, dout), lambda i: (0, 0)),
            pl.BlockSpec((1, dout), lambda i: (0, 0)),
        ],
        out_specs=pl.BlockSpec((br, dout), lambda i: (i, 0)),
        out_shape=jax.ShapeDtypeStruct((R, dout), _f32),
    )(xv, p["l1"]["W"], p["l1"]["b"].reshape(1, -1),
      p["l2"]["W"], p["l2"]["b"].reshape(1, -1),
      p["skip"]["W"], p["skip"]["b"].reshape(1, -1))


def _edge_scale(i):
    return ((i >= (E // 512)).astype(jnp.int32)
            + (i >= ((E + EMP) // 512)).astype(jnp.int32))


def _edge_mlp_tc(cat, W1, b1, W2, b2):
    def body(c_ref, w1, b1r, w2, b2r, o_ref):
        cb = c_ref[...]
        hh = _leaky(_dot(cb, w1[0]) + b1r[0])
        o_ref[...] = _dot(hh, w2[0]) + b2r[0] + cb[:, 2 * H:]

    wmap = lambda i: (_edge_scale(i), 0, 0)
    return pl.pallas_call(
        body,
        grid=(EP // 512,),
        in_specs=[
            pl.BlockSpec((512, 3 * H), lambda i: (i, 0)),
            pl.BlockSpec((1, 3 * H, H), wmap),
            pl.BlockSpec((1, 1, H), wmap),
            pl.BlockSpec((1, H, H), wmap),
            pl.BlockSpec((1, 1, H), wmap),
        ],
        out_specs=pl.BlockSpec((512, H), lambda i: (i, 0)),
        out_shape=jax.ShapeDtypeStruct((EP, H), _f32),
    )(cat, W1, b1, W2, b2)


def _edge_direct_tc(cat, e1, be1, e2, be2):
    R = cat.shape[0]

    def body(c_ref, w1, b1r, w2, b2r, o_ref):
        cb = c_ref[...]
        hh = _leaky(_dot(cb, w1[...]) + b1r[...])
        o_ref[...] = _dot(hh, w2[...]) + b2r[...] + cb[:, 2 * H:]

    return pl.pallas_call(
        body,
        grid=(R // 512,),
        in_specs=[
            pl.BlockSpec((512, 3 * H), lambda i: (i, 0)),
            pl.BlockSpec((3 * H, H), lambda i: (0, 0)),
            pl.BlockSpec((1, H), lambda i: (0, 0)),
            pl.BlockSpec((H, H), lambda i: (0, 0)),
            pl.BlockSpec((1, H), lambda i: (0, 0)),
        ],
        out_specs=pl.BlockSpec((512, H), lambda i: (i, 0)),
        out_shape=jax.ShapeDtypeStruct((R, H), _f32),
    )(cat, e1, be1.reshape(1, -1), e2, be2.reshape(1, -1))


def _node_tc(cat, n1, bn1, n2, bn2):
    R = cat.shape[0]

    def body(c_ref, w1, b1r, w2, b2r, o_ref):
        cb = c_ref[...]
        hh = _leaky(_dot(cb, w1[...]) + b1r[...])
        o_ref[...] = _dot(hh, w2[...]) + b2r[...] + cb[:, :H]

    return pl.pallas_call(
        body,
        grid=(R // 512,),
        in_specs=[
            pl.BlockSpec((512, 2 * H), lambda i: (i, 0)),
            pl.BlockSpec((2 * H, H), lambda i: (0, 0)),
            pl.BlockSpec((1, H), lambda i: (0, 0)),
            pl.BlockSpec((H, H), lambda i: (0, 0)),
            pl.BlockSpec((1, H), lambda i: (0, 0)),
        ],
        out_specs=pl.BlockSpec((512, H), lambda i: (i, 0)),
        out_shape=jax.ShapeDtypeStruct((R, H), _f32),
    )(cat, n1, bn1.reshape(1, -1), n2, bn2.reshape(1, -1))


def _make_sc_gather(nlists, k):
    per = k * 128
    mesh = plsc.VectorSubcoreMesh(core_axis_name="c", subcore_axis_name="s",
                                  num_cores=NC, num_subcores=NS)

    @functools.partial(
        pl.kernel,
        out_type=jax.ShapeDtypeStruct((nlists, per * NW, H), _f32),
        mesh=mesh,
        scratch_types=[pltpu.VMEM((k, 128), jnp.int32),
                       pltpu.VMEM((per, H), _f32),
                       pltpu.SemaphoreType.DMA],
    )
    def g(table, idx, out, idx_v, rows_v, sem):
        wid = lax.axis_index("s") * NC + lax.axis_index("c")
        base = wid * per
        for l in range(nlists):
            pltpu.sync_copy(idx.at[l, wid], idx_v)
            cps = [pltpu.async_copy(table.at[idx_v.at[j]],
                                    rows_v.at[pl.ds(j * 128, 128)], sem)
                   for j in range(k)]
            for cp in cps:
                cp.wait()
            pltpu.sync_copy(rows_v, out.at[l, pl.ds(base, per)])

    return g


@functools.lru_cache(maxsize=None)
def _sc_kernels():
    return (_make_sc_gather(2, KCH),
            _make_sc_gather(1, 1))


def kernel(x, edge_index, edge_weight, m_map, l_map, m_adj, l_adj, params):
    p = params
    i32 = jnp.int32
    _gather_edges, _gather_final = _sc_kernels()

    def _bd(a, b):
        return jnp.dot(a.astype(jnp.bfloat16), b.astype(jnp.bfloat16),
                       preferred_element_type=_f32)

    def _nupd_x(pp, xd, ag):
        nf = jnp.concatenate([xd, ag], axis=-1)
        nf = _leaky(_bd(nf, pp["n1"]["W"]) + pp["n1"]["b"])
        return _bd(nf, pp["n2"]["W"]) + pp["n2"]["b"] + xd

    def _gcn_x(pp, x_src, x_dst, adj, e):
        s_, d_ = adj[0], adj[1]
        ef = jnp.concatenate([x_src[s_], x_dst[d_], e], axis=-1)
        ef = _leaky(_bd(ef, pp["e1"]["W"]) + pp["e1"]["b"])
        ef = _bd(ef, pp["e2"]["W"]) + pp["e2"]["b"] + e
        agg = jax.ops.segment_sum(ef, d_, num_segments=x_dst.shape[0])
        nf = jnp.concatenate([x_dst, agg], axis=-1)
        nf = _leaky(_bd(nf, pp["n1"]["W"]) + pp["n1"]["b"])
        nf = _bd(nf, pp["n2"]["W"]) + pp["n2"]["b"] + x_dst
        return nf, ef

    s_all = jnp.concatenate([
        edge_index[0],
        m_adj[0] + OFF_M, jnp.full((EMP - EM,), OFF_M, i32),
        l_adj[0] + OFF_L, jnp.full((ELP - EL,), OFF_L, i32)])
    d_all = jnp.concatenate([
        edge_index[1],
        m_adj[1] + OFF_M, jnp.full((EMP - EM,), OFF_M + NMP - 1, i32),
        l_adj[1] + OFF_L, jnp.full((ELP - EL,), OFF_L + NLP - 1, i32)])
    idx_sd = jnp.stack([s_all, d_all]).reshape(2, NW, KCH, 128)

    x2 = x.reshape(N, T * F)
    cm = jnp.maximum(jax.ops.segment_sum(jnp.ones((N,), _f32), m_map,
                                         num_segments=NM), 1.0)
    pooled_m = jax.ops.segment_sum(x2, m_map, num_segments=NM) / cm[:, None]
    cl = jnp.maximum(jax.ops.segment_sum(jnp.ones((N,), _f32), l_map,
                                         num_segments=NL), 1.0)
    pooled_l = jax.ops.segment_sum(x2, l_map, num_segments=NL) / cl[:, None]
    mg = _rmlp_tc(_padr(pooled_m, NMP).reshape(NMP * T, F),
                  p["cm_node"]).reshape(NMP, T, H)
    lg = _rmlp_tc(_padr(pooled_l, NLP).reshape(NLP * T, F),
                  p["cl_node"]).reshape(NLP, T, H)
    enc = _rmlp_tc(x.reshape(N * T, F), p["enc"]).reshape(N, T, H)

    e_fine = _rmlp_tc(jnp.stack([jnp.sin(edge_weight), jnp.cos(edge_weight)],
                                axis=-1), p["attr"])
    m_ce = _rmlp_tc(_padr(p["cm_edge_raw"], EMP), p["cm_edge"])
    l_ce = _rmlp_tc(_padr(p["cl_edge_raw"], ELP), p["cl_edge"])
    e_table = jnp.concatenate([e_fine, m_ce, l_ce])

    s_e = _rmlp_tc(p["s_edge_raw"], p["st"])
    m_e = _rmlp_tc(_padr(p["m_edge_raw"], NMP), p["mt"])[:NM]
    l_e = _rmlp_tc(_padr(p["l_edge_raw"], NLP), p["lt"])[:NL]

    sg_c, sg_n = enc[:, 0], enc[:, 1]
    mg_c, mg_n = mg[:NM, 0], mg[:NM, 1]
    lg_c, lg_n = lg[:NL, 0], lg[:NL, 1]
    m_tadj = jnp.stack([jnp.arange(NM), jnp.arange(NM)])
    l_tadj = jnp.stack([jnp.arange(NL), jnp.arange(NL)])

    lw = []
    for i in range(NLAYERS):
        lw.append(tuple(
            jnp.stack([p[g][i][nm]["W"] for g in ("fine", "med", "coa")])
            if kind == "W" else
            jnp.stack([p[g][i][nm]["b"] for g in ("fine", "med", "coa")]
                      ).reshape(3, 1, H)
            for nm, kind in (("e1", "W"), ("e1", "b"),
                             ("e2", "W"), ("e2", "b"))))

    for _step in range(T - 1):
        for i in range(NLAYERS):
            W1, b1, W2, b2 = lw[i]
            c_table = jnp.concatenate(
                [sg_c, _padr(mg_c, NMP), _padr(lg_c, NLP)])
            g2 = _gather_edges(c_table, idx_sd)
            ef = _edge_mlp_tc(
                jnp.concatenate([g2[0], g2[1], e_table], axis=1),
                W1, b1, W2, b2)
            agg_f = jax.ops.segment_sum(ef[:E], edge_index[1],
                                        num_segments=N)
            agg_m = jax.ops.segment_sum(ef[E:E + EM], m_adj[1],
                                        num_segments=NM)
            agg_l = jax.ops.segment_sum(ef[E + EMP:E + EMP + EL], l_adj[1],
                                        num_segments=NL)
            sg_c = _node_tc(jnp.concatenate([sg_c, agg_f], axis=1),
                            p["fine"][i]["n1"]["W"], p["fine"][i]["n1"]["b"],
                            p["fine"][i]["n2"]["W"], p["fine"][i]["n2"]["b"])
            mg_c = _nupd_x(p["med"][i], mg_c, agg_m)
            lg_c = _nupd_x(p["coa"][i], lg_c, agg_l)
            e_table = ef

        st = p["s_temp"]
        s_e = _edge_direct_tc(jnp.concatenate([sg_c, sg_n, s_e], axis=1),
                              st["e1"]["W"], st["e1"]["b"],
                              st["e2"]["W"], st["e2"]["b"])
        sg_c = _node_tc(jnp.concatenate([sg_n, s_e], axis=1),
                        st["n1"]["W"], st["n1"]["b"],
                        st["n2"]["W"], st["n2"]["b"])
        mg_c, m_e = _gcn_x(p["m_temp"], mg_c, mg_n, m_tadj, m_e)
        lg_c, l_e = _gcn_x(p["l_temp"], lg_c, lg_n, l_tadj, l_e)

    dec_m = _rmlp_tc(_padr(mg_c, 1024), p["dec_m"])[:NMP]
    dec_l = _rmlp_tc(_padr(lg_c, 512), p["dec_l"])[:NLP]
    table_f = jnp.concatenate([dec_m, dec_l])
    idx_f = jnp.concatenate([m_map, l_map + NMP]).reshape(1, NW, 1, 128)
    gf = _gather_final(table_f, idx_f)[0]
    xcat = jnp.concatenate([sg_c, gf[:N], gf[N:]], axis=-1)
    out = _rmlp_tc(xcat, p["readout"])
    return out[:, None, :]


# speedup vs baseline: 1.2788x; 1.2788x over previous
import functools

import jax
import jax.numpy as jnp
from jax import lax
from jax.experimental import pallas as pl
from jax.experimental.pallas import tpu as pltpu
from jax.experimental.pallas import tpu_sc as plsc

N = 2048
T = 3
F = 26
H = 128
E = 16384
NM = 842
EM = 6736
NL = 122
EL = 976
NLAYERS = 4
OUT = 26

NMP = 896
NLP = 128
OFF_M = N
OFF_L = N + NMP
NP = N + NMP + NLP
EMP = 7168
ELP = 1024
EP = E + EMP + ELP

NC = 2
NS = 16
NW = NC * NS
EPW = EP // NW
KCH = EPW // 128

_f32 = jnp.float32


def _leaky(x):
    return jnp.where(x >= 0, x, 0.01 * x)


def _dot(a, b):
    return jnp.dot(a.astype(jnp.bfloat16), b.astype(jnp.bfloat16),
                   preferred_element_type=_f32)


def _padr(a, rows):
    pad = [(0, rows - a.shape[0])] + [(0, 0)] * (a.ndim - 1)
    return jnp.pad(a, pad)


def _rmlp_tc(xv, p):
    if xv.shape[1] < 8:
        pk = 128 - xv.shape[1]
        xv = jnp.pad(xv, ((0, 0), (0, pk)))
        p = dict(p,
                 l1={"W": jnp.pad(p["l1"]["W"], ((0, pk), (0, 0))),
                     "b": p["l1"]["b"]},
                 skip={"W": jnp.pad(p["skip"]["W"], ((0, pk), (0, 0))),
                       "b": p["skip"]["b"]})
    R, din = xv.shape
    h = p["l1"]["W"].shape[1]
    dout = p["l2"]["W"].shape[1]
    br = 512 if R % 512 == 0 else 128

    def body(x_ref, w1, b1, w2, b2, ws, bs, o_ref):
        xb = x_ref[...]
        hh = _leaky(_dot(xb, w1[...]) + b1[...])
        o_ref[...] = (_dot(hh, w2[...]) + b2[...]
                      + _dot(xb, ws[...]) + bs[...])

    return pl.pallas_call(
        body,
        grid=(R // br,),
        in_specs=[
            pl.BlockSpec((br, din), lambda i: (i, 0)),
            pl.BlockSpec((din, h), lambda i: (0, 0)),
            pl.BlockSpec((1, h), lambda i: (0, 0)),
            pl.BlockSpec((h, dout), lambda i: (0, 0)),
            pl.BlockSpec((1, dout), lambda i: (0, 0)),
            pl.BlockSpec((din, dout), lambda i: (0, 0)),
            pl.BlockSpec((1, dout), lambda i: (0, 0)),
        ],
        out_specs=pl.BlockSpec((br, dout), lambda i: (i, 0)),
        out_shape=jax.ShapeDtypeStruct((R, dout), _f32),
    )(xv, p["l1"]["W"], p["l1"]["b"].reshape(1, -1),
      p["l2"]["W"], p["l2"]["b"].reshape(1, -1),
      p["skip"]["W"], p["skip"]["b"].reshape(1, -1))


def _edge_scale(i):
    return ((i >= (E // 512)).astype(jnp.int32)
            + (i >= ((E + EMP) // 512)).astype(jnp.int32))


def _edge_mlp_tc(cat, W1, b1, W2, b2):
    def body(c_ref, w1, b1r, w2, b2r, o_ref):
        cb = c_ref[...]
        hh = _leaky(_dot(cb, w1[0]) + b1r[0])
        o_ref[...] = _dot(hh, w2[0]) + b2r[0] + cb[:, 2 * H:]

    wmap = lambda i: (_edge_scale(i), 0, 0)
    return pl.pallas_call(
        body,
        grid=(EP // 512,),
        in_specs=[
            pl.BlockSpec((512, 3 * H), lambda i: (i, 0)),
            pl.BlockSpec((1, 3 * H, H), wmap),
            pl.BlockSpec((1, 1, H), wmap),
            pl.BlockSpec((1, H, H), wmap),
            pl.BlockSpec((1, 1, H), wmap),
        ],
        out_specs=pl.BlockSpec((512, H), lambda i: (i, 0)),
        out_shape=jax.ShapeDtypeStruct((EP, H), _f32),
    )(cat, W1, b1, W2, b2)


def _edge_direct_tc(cat, e1, be1, e2, be2):
    R = cat.shape[0]

    def body(c_ref, w1, b1r, w2, b2r, o_ref):
        cb = c_ref[...]
        hh = _leaky(_dot(cb, w1[...]) + b1r[...])
        o_ref[...] = _dot(hh, w2[...]) + b2r[...] + cb[:, 2 * H:]

    return pl.pallas_call(
        body,
        grid=(R // 512,),
        in_specs=[
            pl.BlockSpec((512, 3 * H), lambda i: (i, 0)),
            pl.BlockSpec((3 * H, H), lambda i: (0, 0)),
            pl.BlockSpec((1, H), lambda i: (0, 0)),
            pl.BlockSpec((H, H), lambda i: (0, 0)),
            pl.BlockSpec((1, H), lambda i: (0, 0)),
        ],
        out_specs=pl.BlockSpec((512, H), lambda i: (i, 0)),
        out_shape=jax.ShapeDtypeStruct((R, H), _f32),
    )(cat, e1, be1.reshape(1, -1), e2, be2.reshape(1, -1))


def _node_tc(cat, n1, bn1, n2, bn2):
    R = cat.shape[0]

    def body(c_ref, w1, b1r, w2, b2r, o_ref):
        cb = c_ref[...]
        hh = _leaky(_dot(cb, w1[...]) + b1r[...])
        o_ref[...] = _dot(hh, w2[...]) + b2r[...] + cb[:, :H]

    return pl.pallas_call(
        body,
        grid=(R // 512,),
        in_specs=[
            pl.BlockSpec((512, 2 * H), lambda i: (i, 0)),
            pl.BlockSpec((2 * H, H), lambda i: (0, 0)),
            pl.BlockSpec((1, H), lambda i: (0, 0)),
            pl.BlockSpec((H, H), lambda i: (0, 0)),
            pl.BlockSpec((1, H), lambda i: (0, 0)),
        ],
        out_specs=pl.BlockSpec((512, H), lambda i: (i, 0)),
        out_shape=jax.ShapeDtypeStruct((R, H), _f32),
    )(cat, n1, bn1.reshape(1, -1), n2, bn2.reshape(1, -1))


def _make_sc_gather(nlists, k):
    per = k * 128
    mesh = plsc.VectorSubcoreMesh(core_axis_name="c", subcore_axis_name="s",
                                  num_cores=NC, num_subcores=NS)

    @functools.partial(
        pl.kernel,
        out_type=jax.ShapeDtypeStruct((nlists, per * NW, H), _f32),
        mesh=mesh,
        scratch_types=[pltpu.VMEM((k, 128), jnp.int32),
                       pltpu.VMEM((per, H), _f32),
                       pltpu.SemaphoreType.DMA],
    )
    def g(table, idx, out, idx_v, rows_v, sem):
        wid = lax.axis_index("s") * NC + lax.axis_index("c")
        base = wid * per
        for l in range(nlists):
            pltpu.sync_copy(idx.at[l, wid], idx_v)
            cps = [pltpu.async_copy(table.at[idx_v.at[j]],
                                    rows_v.at[pl.ds(j * 128, 128)], sem)
                   for j in range(k)]
            for cp in cps:
                cp.wait()
            pltpu.sync_copy(rows_v, out.at[l, pl.ds(base, per)])

    return g


@functools.lru_cache(maxsize=None)
def _sc_kernels():
    return (_make_sc_gather(2, KCH),
            _make_sc_gather(1, 1))


def kernel(x, edge_index, edge_weight, m_map, l_map, m_adj, l_adj, params):
    p = params
    i32 = jnp.int32
    _gather_edges, _gather_final = _sc_kernels()

    def _bd(a, b):
        return jnp.dot(a.astype(jnp.bfloat16), b.astype(jnp.bfloat16),
                       preferred_element_type=_f32)

    def _nupd_x(pp, xd, ag):
        nf = jnp.concatenate([xd, ag], axis=-1)
        nf = _leaky(_bd(nf, pp["n1"]["W"]) + pp["n1"]["b"])
        return _bd(nf, pp["n2"]["W"]) + pp["n2"]["b"] + xd

    def _gcn_ident_x(pp, x_src, x_dst, e):
        ef = jnp.concatenate([x_src, x_dst, e], axis=-1)
        ef = _leaky(_bd(ef, pp["e1"]["W"]) + pp["e1"]["b"])
        ef = _bd(ef, pp["e2"]["W"]) + pp["e2"]["b"] + e
        nf = jnp.concatenate([x_dst, ef], axis=-1)
        nf = _leaky(_bd(nf, pp["n1"]["W"]) + pp["n1"]["b"])
        nf = _bd(nf, pp["n2"]["W"]) + pp["n2"]["b"] + x_dst
        return nf, ef

    s_all = jnp.concatenate([
        edge_index[0],
        m_adj[0] + OFF_M, jnp.full((EMP - EM,), OFF_M, i32),
        l_adj[0] + OFF_L, jnp.full((ELP - EL,), OFF_L, i32)])
    d_all = jnp.concatenate([
        edge_index[1],
        m_adj[1] + OFF_M, jnp.full((EMP - EM,), OFF_M + NMP - 1, i32),
        l_adj[1] + OFF_L, jnp.full((ELP - EL,), OFF_L + NLP - 1, i32)])
    idx_sd = jnp.stack([s_all, d_all]).reshape(2, NW, KCH, 128)

    x2 = x.reshape(N, T * F)
    cm = jnp.maximum(jax.ops.segment_sum(jnp.ones((N,), _f32), m_map,
                                         num_segments=NM), 1.0)
    pooled_m = jax.ops.segment_sum(x2, m_map, num_segments=NM) / cm[:, None]
    cl = jnp.maximum(jax.ops.segment_sum(jnp.ones((N,), _f32), l_map,
                                         num_segments=NL), 1.0)
    pooled_l = jax.ops.segment_sum(x2, l_map, num_segments=NL) / cl[:, None]
    mg = _rmlp_tc(_padr(pooled_m, NMP).reshape(NMP * T, F),
                  p["cm_node"]).reshape(NMP, T, H)
    lg = _rmlp_tc(_padr(pooled_l, NLP).reshape(NLP * T, F),
                  p["cl_node"]).reshape(NLP, T, H)
    enc = _rmlp_tc(x.reshape(N * T, F), p["enc"]).reshape(N, T, H)

    e_fine = _rmlp_tc(jnp.stack([jnp.sin(edge_weight), jnp.cos(edge_weight)],
                                axis=-1), p["attr"])
    m_ce = _rmlp_tc(_padr(p["cm_edge_raw"], EMP), p["cm_edge"])
    l_ce = _rmlp_tc(_padr(p["cl_edge_raw"], ELP), p["cl_edge"])
    e_table = jnp.concatenate([e_fine, m_ce, l_ce])

    s_e = _rmlp_tc(p["s_edge_raw"], p["st"])
    m_e = _rmlp_tc(_padr(p["m_edge_raw"], NMP), p["mt"])[:NM]
    l_e = _rmlp_tc(_padr(p["l_edge_raw"], NLP), p["lt"])[:NL]

    sg_c, sg_n = enc[:, 0], enc[:, 1]
    mg_c, mg_n = mg[:NM, 0], mg[:NM, 1]
    lg_c, lg_n = lg[:NL, 0], lg[:NL, 1]

    lw = []
    for i in range(NLAYERS):
        lw.append(tuple(
            jnp.stack([p[g][i][nm]["W"] for g in ("fine", "med", "coa")])
            if kind == "W" else
            jnp.stack([p[g][i][nm]["b"] for g in ("fine", "med", "coa")]
                      ).reshape(3, 1, H)
            for nm, kind in (("e1", "W"), ("e1", "b"),
                             ("e2", "W"), ("e2", "b"))))

    for _step in range(T - 1):
        for i in range(NLAYERS):
            W1, b1, W2, b2 = lw[i]
            c_table = jnp.concatenate(
                [sg_c, _padr(mg_c, NMP), _padr(lg_c, NLP)])
            g2 = _gather_edges(c_table, idx_sd)
            ef = _edge_mlp_tc(
                jnp.concatenate([g2[0], g2[1], e_table], axis=1),
                W1, b1, W2, b2)
            agg_all = jax.ops.segment_sum(ef, d_all, num_segments=NP)
            agg_f = agg_all[:N]
            agg_m = agg_all[OFF_M:OFF_M + NM]
            agg_l = agg_all[OFF_L:OFF_L + NL]
            sg_c = _node_tc(jnp.concatenate([sg_c, agg_f], axis=1),
                            p["fine"][i]["n1"]["W"], p["fine"][i]["n1"]["b"],
                            p["fine"][i]["n2"]["W"], p["fine"][i]["n2"]["b"])
            mg_c = _nupd_x(p["med"][i], mg_c, agg_m)
            lg_c = _nupd_x(p["coa"][i], lg_c, agg_l)
            e_table = ef

        st = p["s_temp"]
        s_e = _edge_direct_tc(jnp.concatenate([sg_c, sg_n, s_e], axis=1),
                              st["e1"]["W"], st["e1"]["b"],
                              st["e2"]["W"], st["e2"]["b"])
        sg_c = _node_tc(jnp.concatenate([sg_n, s_e], axis=1),
                        st["n1"]["W"], st["n1"]["b"],
                        st["n2"]["W"], st["n2"]["b"])
        mg_c, m_e = _gcn_ident_x(p["m_temp"], mg_c, mg_n, m_e)
        lg_c, l_e = _gcn_ident_x(p["l_temp"], lg_c, lg_n, l_e)

    dec_m = _rmlp_tc(_padr(mg_c, 1024), p["dec_m"])[:NMP]
    dec_l = _rmlp_tc(_padr(lg_c, 512), p["dec_l"])[:NLP]
    table_f = jnp.concatenate([dec_m, dec_l])
    idx_f = jnp.concatenate([m_map, l_map + NMP]).reshape(1, NW, 1, 128)
    gf = _gather_final(table_f, idx_f)[0]
    xcat = jnp.concatenate([sg_c, gf[:N], gf[N:]], axis=-1)
    out = _rmlp_tc(xcat, p["readout"])
    return out[:, None, :]


# speedup vs baseline: 1.3720x; 1.0729x over previous
import functools

import jax
import jax.numpy as jnp
from jax import lax
from jax.experimental import pallas as pl
from jax.experimental.pallas import tpu as pltpu
from jax.experimental.pallas import tpu_sc as plsc

N = 2048
T = 3
F = 26
H = 128
E = 16384
NM = 842
EM = 6736
NL = 122
EL = 976
NLAYERS = 4
OUT = 26

NMP = 896
NLP = 128
OFF_M = N
OFF_L = N + NMP
NP = N + NMP + NLP
EMP = 7168
ELP = 1024
EP = E + EMP + ELP

NC = 2
NS = 16
NW = NC * NS
EPW = EP // NW
KCH = EPW // 128

_f32 = jnp.float32


def _leaky(x):
    return jnp.where(x >= 0, x, 0.01 * x)


def _dot(a, b):
    return jnp.dot(a.astype(jnp.bfloat16), b.astype(jnp.bfloat16),
                   preferred_element_type=_f32)


def _padr(a, rows):
    pad = [(0, rows - a.shape[0])] + [(0, 0)] * (a.ndim - 1)
    return jnp.pad(a, pad)


def _rmlp_tc(xv, p):
    if xv.shape[1] < 8:
        pk = 128 - xv.shape[1]
        xv = jnp.pad(xv, ((0, 0), (0, pk)))
        p = dict(p,
                 l1={"W": jnp.pad(p["l1"]["W"], ((0, pk), (0, 0))),
                     "b": p["l1"]["b"]},
                 skip={"W": jnp.pad(p["skip"]["W"], ((0, pk), (0, 0))),
                       "b": p["skip"]["b"]})
    R, din = xv.shape
    h = p["l1"]["W"].shape[1]
    dout = p["l2"]["W"].shape[1]
    br = 512 if R % 512 == 0 else 128

    def body(x_ref, w1, b1, w2, b2, ws, bs, o_ref):
        xb = x_ref[...]
        hh = _leaky(_dot(xb, w1[...]) + b1[...])
        o_ref[...] = (_dot(hh, w2[...]) + b2[...]
                      + _dot(xb, ws[...]) + bs[...])

    return pl.pallas_call(
        body,
        grid=(R // br,),
        in_specs=[
            pl.BlockSpec((br, din), lambda i: (i, 0)),
            pl.BlockSpec((din, h), lambda i: (0, 0)),
            pl.BlockSpec((1, h), lambda i: (0, 0)),
            pl.BlockSpec((h, dout), lambda i: (0, 0)),
            pl.BlockSpec((1, dout), lambda i: (0, 0)),
            pl.BlockSpec((din, dout), lambda i: (0, 0)),
            pl.BlockSpec((1, dout), lambda i: (0, 0)),
        ],
        out_specs=pl.BlockSpec((br, dout), lambda i: (i, 0)),
        out_shape=jax.ShapeDtypeStruct((R, dout), _f32),
    )(xv, p["l1"]["W"], p["l1"]["b"].reshape(1, -1),
      p["l2"]["W"], p["l2"]["b"].reshape(1, -1),
      p["skip"]["W"], p["skip"]["b"].reshape(1, -1))


def _edge_scale(i):
    return ((i >= (E // 512)).astype(jnp.int32)
            + (i >= ((E + EMP) // 512)).astype(jnp.int32))


def _edge_mlp_tc(gs, gd, ev, W1, b1, W2, b2):
    def body(gs_ref, gd_ref, e_ref, w1, b1r, w2, b2r, o_ref):
        eb = e_ref[...]
        cb = jnp.concatenate([gs_ref[...], gd_ref[...], eb], axis=1)
        hh = _leaky(_dot(cb, w1[0]) + b1r[0])
        o_ref[...] = _dot(hh, w2[0]) + b2r[0] + eb

    wmap = lambda i: (_edge_scale(i), 0, 0)
    rmap = lambda i: (i, 0)
    return pl.pallas_call(
        body,
        grid=(EP // 512,),
        in_specs=[
            pl.BlockSpec((512, H), rmap),
            pl.BlockSpec((512, H), rmap),
            pl.BlockSpec((512, H), rmap),
            pl.BlockSpec((1, 3 * H, H), wmap),
            pl.BlockSpec((1, 1, H), wmap),
            pl.BlockSpec((1, H, H), wmap),
            pl.BlockSpec((1, 1, H), wmap),
        ],
        out_specs=pl.BlockSpec((512, H), lambda i: (i, 0)),
        out_shape=jax.ShapeDtypeStruct((EP, H), _f32),
    )(gs, gd, ev, W1, b1, W2, b2)


def _edge_direct_tc(cat, e1, be1, e2, be2):
    R = cat.shape[0]

    def body(c_ref, w1, b1r, w2, b2r, o_ref):
        cb = c_ref[...]
        hh = _leaky(_dot(cb, w1[...]) + b1r[...])
        o_ref[...] = _dot(hh, w2[...]) + b2r[...] + cb[:, 2 * H:]

    return pl.pallas_call(
        body,
        grid=(R // 512,),
        in_specs=[
            pl.BlockSpec((512, 3 * H), lambda i: (i, 0)),
            pl.BlockSpec((3 * H, H), lambda i: (0, 0)),
            pl.BlockSpec((1, H), lambda i: (0, 0)),
            pl.BlockSpec((H, H), lambda i: (0, 0)),
            pl.BlockSpec((1, H), lambda i: (0, 0)),
        ],
        out_specs=pl.BlockSpec((512, H), lambda i: (i, 0)),
        out_shape=jax.ShapeDtypeStruct((R, H), _f32),
    )(cat, e1, be1.reshape(1, -1), e2, be2.reshape(1, -1))


def _node_tc(cat, n1, bn1, n2, bn2):
    R = cat.shape[0]

    def body(c_ref, w1, b1r, w2, b2r, o_ref):
        cb = c_ref[...]
        hh = _leaky(_dot(cb, w1[...]) + b1r[...])
        o_ref[...] = _dot(hh, w2[...]) + b2r[...] + cb[:, :H]

    return pl.pallas_call(
        body,
        grid=(R // 512,),
        in_specs=[
            pl.BlockSpec((512, 2 * H), lambda i: (i, 0)),
            pl.BlockSpec((2 * H, H), lambda i: (0, 0)),
            pl.BlockSpec((1, H), lambda i: (0, 0)),
            pl.BlockSpec((H, H), lambda i: (0, 0)),
            pl.BlockSpec((1, H), lambda i: (0, 0)),
        ],
        out_specs=pl.BlockSpec((512, H), lambda i: (i, 0)),
        out_shape=jax.ShapeDtypeStruct((R, H), _f32),
    )(cat, n1, bn1.reshape(1, -1), n2, bn2.reshape(1, -1))


def _make_sc_gather(nlists, k):
    per = k * 128
    mesh = plsc.VectorSubcoreMesh(core_axis_name="c", subcore_axis_name="s",
                                  num_cores=NC, num_subcores=NS)

    nh = max(k // 2, 1)
    half = nh * 128

    @functools.partial(
        pl.kernel,
        out_type=jax.ShapeDtypeStruct((nlists, per * NW, H), _f32),
        mesh=mesh,
        scratch_types=[pltpu.VMEM((k, 128), jnp.int32),
                       pltpu.VMEM((half, H), _f32),
                       pltpu.VMEM((half, H), _f32),
                       pltpu.SemaphoreType.DMA,
                       pltpu.SemaphoreType.DMA],
    )
    def g(table, idx, out, idx_v, rows_a, rows_b, sem_g, sem_w):
        wid = lax.axis_index("s") * NC + lax.axis_index("c")
        base = wid * per
        bufs = (rows_a, rows_b)
        nhalves = (k + nh - 1) // nh
        writes = []
        for l in range(nlists):
            pltpu.sync_copy(idx.at[l, wid], idx_v)
            for hh in range(nhalves):
                t = l * nhalves + hh
                buf = bufs[t % 2]
                if t >= 2:
                    writes[t - 2].wait()
                cps = [pltpu.async_copy(table.at[idx_v.at[hh * nh + j]],
                                        buf.at[pl.ds(j * 128, 128)], sem_g)
                       for j in range(nh)]
                for cp in cps:
                    cp.wait()
                writes.append(pltpu.async_copy(
                    buf, out.at[l, pl.ds(base + hh * half, half)], sem_w))
        for w in writes[-2:]:
            w.wait()

    return g


@functools.lru_cache(maxsize=None)
def _sc_kernels():
    return (_make_sc_gather(2, KCH),
            _make_sc_gather(1, 1))


def kernel(x, edge_index, edge_weight, m_map, l_map, m_adj, l_adj, params):
    p = params
    i32 = jnp.int32
    _gather_edges, _gather_final = _sc_kernels()

    def _bd(a, b):
        return jnp.dot(a.astype(jnp.bfloat16), b.astype(jnp.bfloat16),
                       preferred_element_type=_f32)

    def _nupd_x(pp, xd, ag):
        nf = jnp.concatenate([xd, ag], axis=-1)
        nf = _leaky(_bd(nf, pp["n1"]["W"]) + pp["n1"]["b"])
        return _bd(nf, pp["n2"]["W"]) + pp["n2"]["b"] + xd

    def _gcn_ident_x(pp, x_src, x_dst, e):
        ef = jnp.concatenate([x_src, x_dst, e], axis=-1)
        ef = _leaky(_bd(ef, pp["e1"]["W"]) + pp["e1"]["b"])
        ef = _bd(ef, pp["e2"]["W"]) + pp["e2"]["b"] + e
        nf = jnp.concatenate([x_dst, ef], axis=-1)
        nf = _leaky(_bd(nf, pp["n1"]["W"]) + pp["n1"]["b"])
        nf = _bd(nf, pp["n2"]["W"]) + pp["n2"]["b"] + x_dst
        return nf, ef

    s_all = jnp.concatenate([
        edge_index[0],
        m_adj[0] + OFF_M, jnp.full((EMP - EM,), OFF_M, i32),
        l_adj[0] + OFF_L, jnp.full((ELP - EL,), OFF_L, i32)])
    d_all = jnp.concatenate([
        edge_index[1],
        m_adj[1] + OFF_M, jnp.full((EMP - EM,), OFF_M + NMP - 1, i32),
        l_adj[1] + OFF_L, jnp.full((ELP - EL,), OFF_L + NLP - 1, i32)])
    idx_sd = jnp.stack([s_all, d_all]).reshape(2, NW, KCH, 128)

    x2 = x.reshape(N, T * F)
    cm = jnp.maximum(jax.ops.segment_sum(jnp.ones((N,), _f32), m_map,
                                         num_segments=NM), 1.0)
    pooled_m = jax.ops.segment_sum(x2, m_map, num_segments=NM) / cm[:, None]
    cl = jnp.maximum(jax.ops.segment_sum(jnp.ones((N,), _f32), l_map,
                                         num_segments=NL), 1.0)
    pooled_l = jax.ops.segment_sum(x2, l_map, num_segments=NL) / cl[:, None]
    mg = _rmlp_tc(_padr(pooled_m, NMP).reshape(NMP * T, F),
                  p["cm_node"]).reshape(NMP, T, H)
    lg = _rmlp_tc(_padr(pooled_l, NLP).reshape(NLP * T, F),
                  p["cl_node"]).reshape(NLP, T, H)
    enc = _rmlp_tc(x.reshape(N * T, F), p["enc"]).reshape(N, T, H)

    e_fine = _rmlp_tc(jnp.stack([jnp.sin(edge_weight), jnp.cos(edge_weight)],
                                axis=-1), p["attr"])
    m_ce = _rmlp_tc(_padr(p["cm_edge_raw"], EMP), p["cm_edge"])
    l_ce = _rmlp_tc(_padr(p["cl_edge_raw"], ELP), p["cl_edge"])
    e_table = jnp.concatenate([e_fine, m_ce, l_ce])

    s_e = _rmlp_tc(p["s_edge_raw"], p["st"])
    m_e = _rmlp_tc(_padr(p["m_edge_raw"], NMP), p["mt"])[:NM]
    l_e = _rmlp_tc(_padr(p["l_edge_raw"], NLP), p["lt"])[:NL]

    sg_c, sg_n = enc[:, 0], enc[:, 1]
    mg_c, mg_n = mg[:NM, 0], mg[:NM, 1]
    lg_c, lg_n = lg[:NL, 0], lg[:NL, 1]

    lw = []
    for i in range(NLAYERS):
        lw.append(tuple(
            jnp.stack([p[g][i][nm]["W"] for g in ("fine", "med", "coa")])
            if kind == "W" else
            jnp.stack([p[g][i][nm]["b"] for g in ("fine", "med", "coa")]
                      ).reshape(3, 1, H)
            for nm, kind in (("e1", "W"), ("e1", "b"),
                             ("e2", "W"), ("e2", "b"))))

    for _step in range(T - 1):
        for i in range(NLAYERS):
            W1, b1, W2, b2 = lw[i]
            c_table = jnp.concatenate(
                [sg_c, _padr(mg_c, NMP), _padr(lg_c, NLP)])
            g2 = _gather_edges(c_table, idx_sd)
            ef = _edge_mlp_tc(g2[0], g2[1], e_table, W1, b1, W2, b2)
            agg_all = jax.ops.segment_sum(ef, d_all, num_segments=NP)
            agg_f = agg_all[:N]
            agg_m = agg_all[OFF_M:OFF_M + NM]
            agg_l = agg_all[OFF_L:OFF_L + NL]
            sg_c = _node_tc(jnp.concatenate([sg_c, agg_f], axis=1),
                            p["fine"][i]["n1"]["W"], p["fine"][i]["n1"]["b"],
                            p["fine"][i]["n2"]["W"], p["fine"][i]["n2"]["b"])
            mg_c = _nupd_x(p["med"][i], mg_c, agg_m)
            lg_c = _nupd_x(p["coa"][i], lg_c, agg_l)
            e_table = ef

        st = p["s_temp"]
        s_e = _edge_direct_tc(jnp.concatenate([sg_c, sg_n, s_e], axis=1),
                              st["e1"]["W"], st["e1"]["b"],
                              st["e2"]["W"], st["e2"]["b"])
        sg_c = _node_tc(jnp.concatenate([sg_n, s_e], axis=1),
                        st["n1"]["W"], st["n1"]["b"],
                        st["n2"]["W"], st["n2"]["b"])
        mg_c, m_e = _gcn_ident_x(p["m_temp"], mg_c, mg_n, m_e)
        lg_c, l_e = _gcn_ident_x(p["l_temp"], lg_c, lg_n, l_e)

    dec_m = _rmlp_tc(_padr(mg_c, 1024), p["dec_m"])[:NMP]
    dec_l = _rmlp_tc(_padr(lg_c, 512), p["dec_l"])[:NLP]
    table_f = jnp.concatenate([dec_m, dec_l])
    idx_f = jnp.concatenate([m_map, l_map + NMP]).reshape(1, NW, 1, 128)
    gf = _gather_final(table_f, idx_f)[0]
    xcat = jnp.concatenate([sg_c, gf[:N], gf[N:]], axis=-1)
    out = _rmlp_tc(xcat, p["readout"])
    return out[:, None, :]
